# Initial kernel scaffold; baseline (speedup 1.0000x reference)
import jax
import jax.numpy as jnp
from jax.experimental import pallas as pl


def kernel(x, W, b, batch):
    raise NotImplementedError("write your pallas kernel here")


# speedup vs baseline: 4.4954x; 4.4954x over previous
import jax
import jax.numpy as jnp
from jax import lax
from jax.experimental import pallas as pl
from jax.experimental.pallas import tpu as pltpu

N = 32768
D = 512
B = 16
BN = 512
NB = N // BN


def _attn_pool_kernel(x_ref, w_ref, b_ref, batch_ref, out_ref, acc_ref, cnt_ref):
    s = pl.program_id(0)
    nb = pl.num_programs(0) // 2

    @pl.when(s == 0)
    def _init():
        acc_ref[...] = jnp.zeros_like(acc_ref)
        cnt_ref[...] = jnp.zeros_like(cnt_ref)

    bvec = batch_ref[0, 0, :]
    seg_iota = lax.broadcasted_iota(jnp.int32, (BN, B), 1)
    onehot = (bvec[:, None] == seg_iota).astype(jnp.float32)

    @pl.when(s < nb)
    def _phase1():
        xb = x_ref[...]
        logit = jnp.sum(xb * w_ref[0, :][None, :], axis=1, keepdims=True) + b_ref[0]
        weighted = xb * jax.nn.sigmoid(logit)
        acc_ref[...] += jnp.dot(onehot.T, weighted,
                                preferred_element_type=jnp.float32)
        cnt_ref[0, :] += jnp.sum(onehot, axis=0)
        out_ref[...] = xb

    @pl.when(s >= nb)
    def _phase2():
        inv = 1.0 / jnp.maximum(cnt_ref[0, :], 1.0)
        means = acc_ref[...] * inv[:, None]
        out_ref[...] = jnp.dot(onehot, means,
                               preferred_element_type=jnp.float32)


def kernel(x, W, b, batch):
    batch32 = batch.astype(jnp.int32).reshape(NB, 1, BN)
    w_row = W.reshape(1, D)

    grid = (2 * NB,)
    out = pl.pallas_call(
        _attn_pool_kernel,
        grid=grid,
        in_specs=[
            pl.BlockSpec((BN, D), lambda s: (jnp.minimum(s, NB - 1), 0)),
            pl.BlockSpec((1, D), lambda s: (0, 0)),
            pl.BlockSpec(memory_space=pltpu.SMEM),
            pl.BlockSpec((1, 1, BN), lambda s: (lax.rem(s, NB), 0, 0)),
        ],
        out_specs=pl.BlockSpec((BN, D), lambda s: (lax.rem(s, NB), s // NB)),
        out_shape=jax.ShapeDtypeStruct((N, 2 * D), jnp.float32),
        scratch_shapes=[
            pltpu.VMEM((B, D), jnp.float32),
            pltpu.VMEM((1, B), jnp.float32),
        ],
    )(x, w_row, b, batch32)
    return out


# speedup vs baseline: 7.8630x; 1.7491x over previous
import jax
import jax.numpy as jnp
from jax import lax
from jax.experimental import pallas as pl
from jax.experimental.pallas import tpu as pltpu

N = 32768
D = 512
B = 16
BN = 2048
NB = N // BN


def _attn_pool_kernel(x_ref, w_ref, b_ref, batch_ref, out_ref, acc_ref, cnt_ref):
    s = pl.program_id(0)
    nb = pl.num_programs(0) // 2

    @pl.when(s == 0)
    def _init():
        acc_ref[...] = jnp.zeros_like(acc_ref)
        cnt_ref[...] = jnp.zeros_like(cnt_ref)

    bvec = batch_ref[0, 0, :]
    seg_iota = lax.broadcasted_iota(jnp.int32, (BN, B), 1)
    onehot = (bvec[:, None] == seg_iota).astype(jnp.float32)

    @pl.when(s < nb)
    def _phase1():
        xb = x_ref[...]
        logit = jnp.sum(xb * w_ref[0, :][None, :], axis=1, keepdims=True) + b_ref[0]
        weighted = xb * jax.nn.sigmoid(logit)
        acc_ref[...] += jnp.dot(onehot.T, weighted,
                                preferred_element_type=jnp.float32)
        cnt_ref[0, :] += jnp.sum(onehot, axis=0)
        out_ref[...] = xb

    @pl.when(s >= nb)
    def _phase2():
        inv = 1.0 / jnp.maximum(cnt_ref[0, :], 1.0)
        means = acc_ref[...] * inv[:, None]
        out_ref[...] = jnp.dot(onehot, means,
                               preferred_element_type=jnp.float32)


def kernel(x, W, b, batch):
    batch32 = batch.astype(jnp.int32).reshape(NB, 1, BN)
    w_row = W.reshape(1, D)

    grid = (2 * NB,)
    out = pl.pallas_call(
        _attn_pool_kernel,
        grid=grid,
        in_specs=[
            pl.BlockSpec((BN, D), lambda s: (jnp.minimum(s, NB - 1), 0)),
            pl.BlockSpec((1, D), lambda s: (0, 0)),
            pl.BlockSpec(memory_space=pltpu.SMEM),
            pl.BlockSpec((1, 1, BN), lambda s: (lax.rem(s, NB), 0, 0)),
        ],
        out_specs=pl.BlockSpec((BN, D), lambda s: (lax.rem(s, NB), s // NB)),
        out_shape=jax.ShapeDtypeStruct((N, 2 * D), jnp.float32),
        scratch_shapes=[
            pltpu.VMEM((B, D), jnp.float32),
            pltpu.VMEM((1, B), jnp.float32),
        ],
    )(x, w_row, b, batch32)
    return out


# speedup vs baseline: 8.3612x; 1.0633x over previous
import jax
import jax.numpy as jnp
from jax import lax
from jax.experimental import pallas as pl
from jax.experimental.pallas import tpu as pltpu

N = 32768
D = 512
B = 16
BN = 4096
NB = N // BN


def _attn_pool_kernel(x_ref, w_ref, b_ref, batch_ref, out_ref, acc_ref, cnt_ref):
    s = pl.program_id(0)
    nb = pl.num_programs(0) // 2

    @pl.when(s == 0)
    def _init():
        acc_ref[...] = jnp.zeros_like(acc_ref)
        cnt_ref[...] = jnp.zeros_like(cnt_ref)

    bvec = batch_ref[0, 0, :]
    seg_iota = lax.broadcasted_iota(jnp.int32, (BN, B), 1)
    onehot = (bvec[:, None] == seg_iota).astype(jnp.float32)

    @pl.when(s < nb)
    def _phase1():
        xb = x_ref[...]
        logit = jnp.sum(xb * w_ref[0, :][None, :], axis=1, keepdims=True) + b_ref[0]
        weighted = xb * jax.nn.sigmoid(logit)
        acc_ref[...] += jnp.dot(onehot.T, weighted,
                                preferred_element_type=jnp.float32)
        cnt_ref[0, :] += jnp.sum(onehot, axis=0)
        out_ref[...] = xb

    @pl.when(s >= nb)
    def _phase2():
        inv = 1.0 / jnp.maximum(cnt_ref[0, :], 1.0)
        means = acc_ref[...] * inv[:, None]
        out_ref[...] = jnp.dot(onehot, means,
                               preferred_element_type=jnp.float32)


def kernel(x, W, b, batch):
    batch32 = batch.astype(jnp.int32).reshape(NB, 1, BN)
    w_row = W.reshape(1, D)

    grid = (2 * NB,)
    out = pl.pallas_call(
        _attn_pool_kernel,
        grid=grid,
        in_specs=[
            pl.BlockSpec((BN, D), lambda s: (jnp.minimum(s, NB - 1), 0)),
            pl.BlockSpec((1, D), lambda s: (0, 0)),
            pl.BlockSpec(memory_space=pltpu.SMEM),
            pl.BlockSpec((1, 1, BN), lambda s: (lax.rem(s, NB), 0, 0)),
        ],
        out_specs=pl.BlockSpec((BN, D), lambda s: (lax.rem(s, NB), s // NB)),
        out_shape=jax.ShapeDtypeStruct((N, 2 * D), jnp.float32),
        scratch_shapes=[
            pltpu.VMEM((B, D), jnp.float32),
            pltpu.VMEM((1, B), jnp.float32),
        ],
    )(x, w_row, b, batch32)
    return out
